# Initial kernel scaffold; baseline (speedup 1.0000x reference)
import jax
import jax.numpy as jnp
from jax.experimental import pallas as pl


def kernel(rays_o, rays_d, bounds, W1, b1, W2, b2):
    raise NotImplementedError("write your pallas kernel here")


# speedup vs baseline: 13.1525x; 13.1525x over previous
import jax
import jax.numpy as jnp
from jax import lax
from jax.experimental import pallas as pl

NRAYS = 4096
NS = 64
NI = 128
NF = NS + NI
HID = 256

TRM1 = 64
TRM2 = 32
TR_S = 64
TR_R = 64

F32 = jnp.float32
I32 = jnp.int32


def _cumsum_lanes(x, n):
    k = 1
    while k < n:
        shifted = jnp.concatenate(
            [jnp.zeros(x.shape[:-1] + (k,), x.dtype), x[..., : n - k]], axis=-1)
        x = x + shifted
        k *= 2
    return x


def _sigmoid(x):
    return 1.0 / (1.0 + jnp.exp(-x))


def _expand_mat(trm, ns):
    blk = trm * ns
    m1 = lax.broadcasted_iota(I32, (trm, blk), 1)
    r0 = lax.broadcasted_iota(I32, (trm, blk), 0)
    return ((m1 >= r0 * ns) & (m1 < (r0 + 1) * ns)).astype(F32)


def _mlp_t(pts_t, w1t, b1c, w2t, b2c):
    h = jnp.maximum(
        jnp.dot(w1t, pts_t, preferred_element_type=F32) + b1c, 0.0)
    return jnp.dot(w2t, h, preferred_element_type=F32) + b2c


def _dot_tl(a, b):
    return lax.dot_general(a, b, (((0,), (0,)), ((), ())),
                           preferred_element_type=F32, precision=lax.Precision.HIGHEST)


def _m1_body(o_ref, d_ref, bnd_ref, t_ref, w1t_ref, b1c_ref, w2t_ref,
             b2c_ref, rawt_ref):
    trm, ns = TRM1, NS
    blk = trm * ns
    et = _expand_mat(trm, ns)
    mlane = lax.broadcasted_iota(I32, (1, blk), 1)
    rem = mlane & (ns - 1)
    i0 = lax.broadcasted_iota(I32, (ns, blk), 0)
    ct = (i0 == rem).astype(F32)

    bnd = bnd_ref[...]
    near_rep = _dot_tl(bnd[:, 0:1], et)
    far_rep = _dot_tl(bnd[:, 1:2], et)
    t_rep = jnp.dot(t_ref[...], ct, preferred_element_type=F32, precision=lax.Precision.HIGHEST)
    zc = near_rep * (1.0 - t_rep) + far_rep * t_rep

    orep = _dot_tl(o_ref[...], et)
    drep = _dot_tl(d_ref[...], et)
    pts_t = orep + drep * zc
    rawt_ref[...] = _mlp_t(pts_t, w1t_ref[...], b1c_ref[...],
                           w2t_ref[...], b2c_ref[...])


def _m2_body(o_ref, d_ref, zct_ref, w1t_ref, b1c_ref, w2t_ref, b2c_ref,
             rawt_ref):
    trm, ns = TRM2, NF
    et = _expand_mat(trm, ns)
    zc = zct_ref[...]
    orep = _dot_tl(o_ref[...], et)
    drep = _dot_tl(d_ref[...], et)
    pts_t = orep + drep * zc
    rawt_ref[...] = _mlp_t(pts_t, w1t_ref[...], b1c_ref[...],
                           w2t_ref[...], b2c_ref[...])


def _render(r0, r1, r2, sg, z, dnorm, tr, ns):
    sigma = jnp.maximum(sg, 0.0)
    dz = z[:, 1:] - z[:, :-1]
    dists = jnp.concatenate([dz, jnp.full((tr, 1), 1e10, F32)], axis=1)
    dists = dists * dnorm
    e = jnp.exp(-sigma * dists)
    alpha = 1.0 - e
    lt = jnp.log(e + 1e-10)
    ct_inc = _cumsum_lanes(lt, ns)
    ct_exc = jnp.concatenate([jnp.zeros((tr, 1), F32), ct_inc[:, :-1]], axis=1)
    trans = jnp.exp(ct_exc)
    weights = alpha * trans
    rgb_cols = [jnp.sum(weights * _sigmoid(rc), axis=1, keepdims=True)
                for rc in (r0, r1, r2)]
    rgb_map = jnp.concatenate(rgb_cols, axis=1)
    return rgb_map, weights, dz


def _s_body(d_ref, bnd_ref, t_ref, u_ref, r0_ref, r1_ref, r2_ref, sg_ref,
            rgbc_ref, zf_ref):
    tr = TR_S
    d = d_ref[...]
    near = bnd_ref[...][:, 0:1]
    far = bnd_ref[...][:, 1:2]
    t = t_ref[...]
    z = near * (1.0 - t) + far * t

    dnorm = jnp.sqrt(jnp.sum(d * d, axis=1, keepdims=True))
    rgb_map, weights, dz = _render(r0_ref[...], r1_ref[...], r2_ref[...],
                                   sg_ref[...], z, dnorm, tr, NS)
    rgbc_ref[...] = rgb_map

    w = weights + 1e-5
    pdf = w / jnp.sum(w, axis=1, keepdims=True)
    cdf = _cumsum_lanes(pdf, NS)
    u = u_ref[...]

    g0 = jnp.zeros((tr, NI), F32)
    g1 = jnp.broadcast_to(cdf[:, 0:1], (tr, NI))
    bb0 = jnp.broadcast_to(z[:, 0:1], (tr, NI))
    bb1 = jnp.broadcast_to(z[:, 1:2], (tr, NI))
    for i in range(1, NS + 1):
        m = (cdf[:, i - 1:i] <= u).astype(F32)
        g0 = g0 + pdf[:, i - 1:i] * m
        if i <= NS - 1:
            g1 = g1 + pdf[:, i:i + 1] * m
            bb0 = bb0 + dz[:, i - 1:i] * m
        if i <= NS - 2:
            bb1 = bb1 + dz[:, i:i + 1] * m
    denom = g1 - g0
    denom = jnp.where(denom < 1e-5, 1.0, denom)
    tt = (u - g0) / denom
    s = bb0 + tt * (bb1 - bb0)

    cnt_z = jnp.zeros((tr, NS), F32)
    for k in range(NI):
        cnt_z = cnt_z + (s[:, k:k + 1] < z).astype(F32)
    cnt_s = jnp.zeros((tr, NI), F32)
    for j in range(NS):
        cnt_s = cnt_s + (z[:, j:j + 1] <= s).astype(F32)
    iz = lax.broadcasted_iota(I32, (1, NS), 1).astype(F32)
    ik = lax.broadcasted_iota(I32, (1, NI), 1).astype(F32)
    rank_z = cnt_z + iz
    rank_s = cnt_s + ik

    pp = lax.broadcasted_iota(I32, (1, NF), 1).astype(F32)
    out = jnp.zeros((tr, NF), F32)
    for i in range(NS):
        out = out + z[:, i:i + 1] * (rank_z[:, i:i + 1] == pp).astype(F32)
    for k in range(NI):
        out = out + s[:, k:k + 1] * (rank_s[:, k:k + 1] == pp).astype(F32)
    zf_ref[...] = out


def _r_body(d_ref, zf_ref, r0_ref, r1_ref, r2_ref, sg_ref,
            rgb_ref, depth_ref, acc_ref, wout_ref):
    tr = TR_R
    d = d_ref[...]
    zf = zf_ref[...]
    dnorm = jnp.sqrt(jnp.sum(d * d, axis=1, keepdims=True))
    rgb_map, weights, _ = _render(r0_ref[...], r1_ref[...], r2_ref[...],
                                  sg_ref[...], zf, dnorm, tr, NF)
    rgb_ref[...] = rgb_map
    depth_ref[...] = jnp.sum(weights * zf, axis=1, keepdims=True)
    acc_ref[...] = jnp.sum(weights, axis=1, keepdims=True)
    wout_ref[...] = weights


@jax.jit
def kernel(rays_o, rays_d, bounds, W1, b1, W2, b2):
    t_vals = jnp.linspace(0.0, 1.0, NS, dtype=F32).reshape(1, NS)
    u_vals = jnp.linspace(0.0, 1.0, NI, dtype=F32).reshape(1, NI)
    w1t = W1.T
    w2t = W2.T
    b1c = b1.reshape(HID, 1)
    b2c = b2.reshape(4, 1)

    m1 = NRAYS * NS
    blk1 = TRM1 * NS
    m2 = NRAYS * NF
    blk2 = TRM2 * NF

    def col_spec(rows, cols):
        return pl.BlockSpec((rows, cols), lambda j: (0, j))

    def fix_spec(shape):
        return pl.BlockSpec(shape, lambda j: (0, 0))

    def ray_spec(tr, cols):
        return pl.BlockSpec((tr, cols), lambda j: (j, 0))

    rawt1 = pl.pallas_call(
        _m1_body,
        grid=(NRAYS // TRM1,),
        in_specs=[
            ray_spec(TRM1, 3), ray_spec(TRM1, 3), ray_spec(TRM1, 2),
            fix_spec((1, NS)), fix_spec((HID, 3)), fix_spec((HID, 1)),
            fix_spec((4, HID)), fix_spec((4, 1)),
        ],
        out_specs=col_spec(4, blk1),
        out_shape=jax.ShapeDtypeStruct((4, m1), F32),
    )(rays_o, rays_d, bounds, t_vals, w1t, b1c, w2t, b2c)
    ch1 = [rawt1[c].reshape(NRAYS, NS) for c in range(4)]

    rgbc, zfine = pl.pallas_call(
        _s_body,
        grid=(NRAYS // TR_S,),
        in_specs=[
            ray_spec(TR_S, 3), ray_spec(TR_S, 2),
            fix_spec((1, NS)), fix_spec((1, NI)),
            ray_spec(TR_S, NS), ray_spec(TR_S, NS), ray_spec(TR_S, NS),
            ray_spec(TR_S, NS),
        ],
        out_specs=[ray_spec(TR_S, 3), ray_spec(TR_S, NF)],
        out_shape=[
            jax.ShapeDtypeStruct((NRAYS, 3), F32),
            jax.ShapeDtypeStruct((NRAYS, NF), F32),
        ],
    )(rays_d, bounds, t_vals, u_vals, *ch1)

    zct = zfine.reshape(1, m2)
    rawt2 = pl.pallas_call(
        _m2_body,
        grid=(NRAYS // TRM2,),
        in_specs=[
            ray_spec(TRM2, 3), ray_spec(TRM2, 3), col_spec(1, blk2),
            fix_spec((HID, 3)), fix_spec((HID, 1)),
            fix_spec((4, HID)), fix_spec((4, 1)),
        ],
        out_specs=col_spec(4, blk2),
        out_shape=jax.ShapeDtypeStruct((4, m2), F32),
    )(rays_o, rays_d, zct, w1t, b1c, w2t, b2c)
    ch2 = [rawt2[c].reshape(NRAYS, NF) for c in range(4)]

    rgb, depth, acc, weights = pl.pallas_call(
        _r_body,
        grid=(NRAYS // TR_R,),
        in_specs=[
            ray_spec(TR_R, 3), ray_spec(TR_R, NF),
            ray_spec(TR_R, NF), ray_spec(TR_R, NF), ray_spec(TR_R, NF),
            ray_spec(TR_R, NF),
        ],
        out_specs=[ray_spec(TR_R, 3), ray_spec(TR_R, 1), ray_spec(TR_R, 1),
                   ray_spec(TR_R, NF)],
        out_shape=[
            jax.ShapeDtypeStruct((NRAYS, 3), F32),
            jax.ShapeDtypeStruct((NRAYS, 1), F32),
            jax.ShapeDtypeStruct((NRAYS, 1), F32),
            jax.ShapeDtypeStruct((NRAYS, NF), F32),
        ],
    )(rays_d, zfine, *ch2)

    return (rgbc, rgb, depth.reshape(NRAYS), acc.reshape(NRAYS), weights)


# speedup vs baseline: 15.2708x; 1.1611x over previous
import jax
import jax.numpy as jnp
from jax import lax
from jax.experimental import pallas as pl

NRAYS = 4096
NS = 64
NI = 128
NF = NS + NI
HID = 256

TRM1 = 64
TRM2 = 32
TR_S = 64
TR_R = 64

F32 = jnp.float32
I32 = jnp.int32


def _cumsum_lanes(x, n):
    k = 1
    while k < n:
        shifted = jnp.concatenate(
            [jnp.zeros(x.shape[:-1] + (k,), x.dtype), x[..., : n - k]], axis=-1)
        x = x + shifted
        k *= 2
    return x


def _sigmoid(x):
    return 1.0 / (1.0 + jnp.exp(-x))


def _mlp_t(pts_t, w1t, b1c, w2t, b2c):
    h = jnp.maximum(
        jnp.dot(w1t, pts_t, preferred_element_type=F32) + b1c, 0.0)
    return jnp.dot(w2t, h, preferred_element_type=F32) + b2c


def _m1_body(orep_ref, drep_ref, bndrep_ref, ttile_ref, w1t_ref, b1c_ref,
             w2t_ref, b2c_ref, rawt_ref):
    near_rep = bndrep_ref[...][0:1, :]
    far_rep = bndrep_ref[...][1:2, :]
    t_rep = ttile_ref[...]
    zc = near_rep * (1.0 - t_rep) + far_rep * t_rep
    pts_t = orep_ref[...] + drep_ref[...] * zc
    rawt_ref[...] = _mlp_t(pts_t, w1t_ref[...], b1c_ref[...],
                           w2t_ref[...], b2c_ref[...])


def _m2_body(orep_ref, drep_ref, zct_ref, w1t_ref, b1c_ref, w2t_ref, b2c_ref,
             rawt_ref):
    pts_t = orep_ref[...] + drep_ref[...] * zct_ref[...]
    rawt_ref[...] = _mlp_t(pts_t, w1t_ref[...], b1c_ref[...],
                           w2t_ref[...], b2c_ref[...])


def _render(r0, r1, r2, sg, z, dnorm, tr, ns):
    sigma = jnp.maximum(sg, 0.0)
    dz = z[:, 1:] - z[:, :-1]
    dists = jnp.concatenate([dz, jnp.full((tr, 1), 1e10, F32)], axis=1)
    dists = dists * dnorm
    e = jnp.exp(-sigma * dists)
    alpha = 1.0 - e
    lt = jnp.log(e + 1e-10)
    ct_inc = _cumsum_lanes(lt, ns)
    ct_exc = jnp.concatenate([jnp.zeros((tr, 1), F32), ct_inc[:, :-1]], axis=1)
    trans = jnp.exp(ct_exc)
    weights = alpha * trans
    rgb_cols = [jnp.sum(weights * _sigmoid(rc), axis=1, keepdims=True)
                for rc in (r0, r1, r2)]
    rgb_map = jnp.concatenate(rgb_cols, axis=1)
    return rgb_map, weights, dz


def _s_body(d_ref, bnd_ref, t_ref, u_ref, r0_ref, r1_ref, r2_ref, sg_ref,
            rgbc_ref, zf_ref):
    tr = TR_S
    d = d_ref[...]
    near = bnd_ref[...][:, 0:1]
    far = bnd_ref[...][:, 1:2]
    t = t_ref[...]
    z = near * (1.0 - t) + far * t

    dnorm = jnp.sqrt(jnp.sum(d * d, axis=1, keepdims=True))
    rgb_map, weights, dz = _render(r0_ref[...], r1_ref[...], r2_ref[...],
                                   sg_ref[...], z, dnorm, tr, NS)
    rgbc_ref[...] = rgb_map

    w = weights + 1e-5
    pdf = w / jnp.sum(w, axis=1, keepdims=True)
    cdf = _cumsum_lanes(pdf, NS)
    u = u_ref[...]

    g0 = jnp.zeros((tr, NI), F32)
    g1 = jnp.broadcast_to(cdf[:, 0:1], (tr, NI))
    bb0 = jnp.broadcast_to(z[:, 0:1], (tr, NI))
    bb1 = jnp.broadcast_to(z[:, 1:2], (tr, NI))
    for i in range(1, NS + 1):
        m = (cdf[:, i - 1:i] <= u).astype(F32)
        g0 = g0 + pdf[:, i - 1:i] * m
        if i <= NS - 1:
            g1 = g1 + pdf[:, i:i + 1] * m
            bb0 = bb0 + dz[:, i - 1:i] * m
        if i <= NS - 2:
            bb1 = bb1 + dz[:, i:i + 1] * m
    denom = g1 - g0
    denom = jnp.where(denom < 1e-5, 1.0, denom)
    tt = (u - g0) / denom
    s = bb0 + tt * (bb1 - bb0)

    cnt_z = jnp.zeros((tr, NS), F32)
    for k in range(NI):
        cnt_z = cnt_z + (s[:, k:k + 1] < z).astype(F32)
    cnt_s = jnp.zeros((tr, NI), F32)
    for j in range(NS):
        cnt_s = cnt_s + (z[:, j:j + 1] <= s).astype(F32)
    iz = lax.broadcasted_iota(I32, (1, NS), 1).astype(F32)
    ik = lax.broadcasted_iota(I32, (1, NI), 1).astype(F32)
    rank_z = cnt_z + iz
    rank_s = cnt_s + ik

    pp = lax.broadcasted_iota(I32, (1, NF), 1).astype(F32)
    out = jnp.zeros((tr, NF), F32)
    for i in range(NS):
        out = out + z[:, i:i + 1] * (rank_z[:, i:i + 1] == pp).astype(F32)
    for k in range(NI):
        out = out + s[:, k:k + 1] * (rank_s[:, k:k + 1] == pp).astype(F32)
    zf_ref[...] = out


def _r_body(d_ref, zf_ref, r0_ref, r1_ref, r2_ref, sg_ref,
            rgb_ref, depth_ref, acc_ref, wout_ref):
    tr = TR_R
    d = d_ref[...]
    zf = zf_ref[...]
    dnorm = jnp.sqrt(jnp.sum(d * d, axis=1, keepdims=True))
    rgb_map, weights, _ = _render(r0_ref[...], r1_ref[...], r2_ref[...],
                                  sg_ref[...], zf, dnorm, tr, NF)
    rgb_ref[...] = rgb_map
    depth_ref[...] = jnp.sum(weights * zf, axis=1, keepdims=True)
    acc_ref[...] = jnp.sum(weights, axis=1, keepdims=True)
    wout_ref[...] = weights


@jax.jit
def kernel(rays_o, rays_d, bounds, W1, b1, W2, b2):
    t_vals = jnp.linspace(0.0, 1.0, NS, dtype=F32).reshape(1, NS)
    u_vals = jnp.linspace(0.0, 1.0, NI, dtype=F32).reshape(1, NI)
    w1t = W1.T
    w2t = W2.T
    b1c = b1.reshape(HID, 1)
    b2c = b2.reshape(4, 1)

    m1 = NRAYS * NS
    blk1 = TRM1 * NS
    m2 = NRAYS * NF
    blk2 = TRM2 * NF

    def col_spec(rows, cols):
        return pl.BlockSpec((rows, cols), lambda j: (0, j))

    def fix_spec(shape):
        return pl.BlockSpec(shape, lambda j: (0, 0))

    def ray_spec(tr, cols):
        return pl.BlockSpec((tr, cols), lambda j: (j, 0))

    orep1 = jnp.repeat(rays_o.T, NS, axis=1)
    drep1 = jnp.repeat(rays_d.T, NS, axis=1)
    bndrep1 = jnp.repeat(bounds.T, NS, axis=1)
    ttile1 = jnp.tile(t_vals, (1, TRM1))
    orep2 = jnp.repeat(rays_o.T, NF, axis=1)
    drep2 = jnp.repeat(rays_d.T, NF, axis=1)

    rawt1 = pl.pallas_call(
        _m1_body,
        grid=(NRAYS // TRM1,),
        in_specs=[
            col_spec(3, blk1), col_spec(3, blk1), col_spec(2, blk1),
            fix_spec((1, blk1)), fix_spec((HID, 3)), fix_spec((HID, 1)),
            fix_spec((4, HID)), fix_spec((4, 1)),
        ],
        out_specs=col_spec(4, blk1),
        out_shape=jax.ShapeDtypeStruct((4, m1), F32),
    )(orep1, drep1, bndrep1, ttile1, w1t, b1c, w2t, b2c)
    ch1 = [rawt1[c].reshape(NRAYS, NS) for c in range(4)]

    rgbc, zfine = pl.pallas_call(
        _s_body,
        grid=(NRAYS // TR_S,),
        in_specs=[
            ray_spec(TR_S, 3), ray_spec(TR_S, 2),
            fix_spec((1, NS)), fix_spec((1, NI)),
            ray_spec(TR_S, NS), ray_spec(TR_S, NS), ray_spec(TR_S, NS),
            ray_spec(TR_S, NS),
        ],
        out_specs=[ray_spec(TR_S, 3), ray_spec(TR_S, NF)],
        out_shape=[
            jax.ShapeDtypeStruct((NRAYS, 3), F32),
            jax.ShapeDtypeStruct((NRAYS, NF), F32),
        ],
    )(rays_d, bounds, t_vals, u_vals, *ch1)

    zct = zfine.reshape(1, m2)
    rawt2 = pl.pallas_call(
        _m2_body,
        grid=(NRAYS // TRM2,),
        in_specs=[
            col_spec(3, blk2), col_spec(3, blk2), col_spec(1, blk2),
            fix_spec((HID, 3)), fix_spec((HID, 1)),
            fix_spec((4, HID)), fix_spec((4, 1)),
        ],
        out_specs=col_spec(4, blk2),
        out_shape=jax.ShapeDtypeStruct((4, m2), F32),
    )(orep2, drep2, zct, w1t, b1c, w2t, b2c)
    ch2 = [rawt2[c].reshape(NRAYS, NF) for c in range(4)]

    rgb, depth, acc, weights = pl.pallas_call(
        _r_body,
        grid=(NRAYS // TR_R,),
        in_specs=[
            ray_spec(TR_R, 3), ray_spec(TR_R, NF),
            ray_spec(TR_R, NF), ray_spec(TR_R, NF), ray_spec(TR_R, NF),
            ray_spec(TR_R, NF),
        ],
        out_specs=[ray_spec(TR_R, 3), ray_spec(TR_R, 1), ray_spec(TR_R, 1),
                   ray_spec(TR_R, NF)],
        out_shape=[
            jax.ShapeDtypeStruct((NRAYS, 3), F32),
            jax.ShapeDtypeStruct((NRAYS, 1), F32),
            jax.ShapeDtypeStruct((NRAYS, 1), F32),
            jax.ShapeDtypeStruct((NRAYS, NF), F32),
        ],
    )(rays_d, zfine, *ch2)

    return (rgbc, rgb, depth.reshape(NRAYS), acc.reshape(NRAYS), weights)


# speedup vs baseline: 18.0697x; 1.1833x over previous
import jax
import jax.numpy as jnp
from jax import lax
from jax.experimental import pallas as pl

NRAYS = 4096
NS = 64
NI = 128
NF = NS + NI
HID = 256

TRM1 = 64
TRM2 = 32
TR_S = 64
TR_R = 64

F32 = jnp.float32
I32 = jnp.int32


def _cumsum_lanes(x, n):
    k = 1
    while k < n:
        shifted = jnp.concatenate(
            [jnp.zeros(x.shape[:-1] + (k,), x.dtype), x[..., : n - k]], axis=-1)
        x = x + shifted
        k *= 2
    return x


def _sigmoid(x):
    return 1.0 / (1.0 + jnp.exp(-x))


def _mlp_t(pts_t, w1t, b1c, w2t, b2c):
    h = jnp.maximum(
        jnp.dot(w1t, pts_t, preferred_element_type=F32) + b1c, 0.0)
    return jnp.dot(w2t, h, preferred_element_type=F32) + b2c


def _m1_body(orep_ref, drep_ref, bndrep_ref, ttile_ref, w1t_ref, b1c_ref,
             w2t_ref, b2c_ref, rawt_ref):
    near_rep = bndrep_ref[...][0:1, :]
    far_rep = bndrep_ref[...][1:2, :]
    t_rep = ttile_ref[...]
    zc = near_rep * (1.0 - t_rep) + far_rep * t_rep
    pts_t = orep_ref[...] + drep_ref[...] * zc
    rawt_ref[...] = _mlp_t(pts_t, w1t_ref[...], b1c_ref[...],
                           w2t_ref[...], b2c_ref[...])


def _m2_body(orep_ref, drep_ref, zct_ref, w1t_ref, b1c_ref, w2t_ref, b2c_ref,
             rawt_ref):
    pts_t = orep_ref[...] + drep_ref[...] * zct_ref[...]
    rawt_ref[...] = _mlp_t(pts_t, w1t_ref[...], b1c_ref[...],
                           w2t_ref[...], b2c_ref[...])


def _render(r0, r1, r2, sg, z, dnorm, tr, ns):
    sigma = jnp.maximum(sg, 0.0)
    dz = z[:, 1:] - z[:, :-1]
    dists = jnp.concatenate([dz, jnp.full((tr, 1), 1e10, F32)], axis=1)
    dists = dists * dnorm
    e = jnp.exp(-sigma * dists)
    alpha = 1.0 - e
    lt = jnp.log(e + 1e-10)
    ct_inc = _cumsum_lanes(lt, ns)
    ct_exc = jnp.concatenate([jnp.zeros((tr, 1), F32), ct_inc[:, :-1]], axis=1)
    trans = jnp.exp(ct_exc)
    weights = alpha * trans
    rgb_cols = [jnp.sum(weights * _sigmoid(rc), axis=1, keepdims=True)
                for rc in (r0, r1, r2)]
    rgb_map = jnp.concatenate(rgb_cols, axis=1)
    return rgb_map, weights, dz


def _s_body(d_ref, bnd_ref, t_ref, u_ref, r0_ref, r1_ref, r2_ref, sg_ref,
            rgbc_ref, zf_ref):
    tr = TR_S
    d = d_ref[...]
    near = bnd_ref[...][:, 0:1]
    far = bnd_ref[...][:, 1:2]
    t = t_ref[...]
    z = near * (1.0 - t) + far * t

    dnorm = jnp.sqrt(jnp.sum(d * d, axis=1, keepdims=True))
    rgb_map, weights, dz = _render(r0_ref[...], r1_ref[...], r2_ref[...],
                                   sg_ref[...], z, dnorm, tr, NS)
    rgbc_ref[...] = rgb_map

    w = weights + 1e-5
    pdf = w / jnp.sum(w, axis=1, keepdims=True)
    cdf = _cumsum_lanes(pdf, NS)
    u = u_ref[...]

    m3 = (cdf[:, :, None] <= u[0][None, None, :]).astype(F32)
    zpad1 = jnp.zeros((tr, 1), F32)
    g1c = jnp.concatenate([pdf[:, 1:], zpad1], axis=1)
    b0c = jnp.concatenate([dz, zpad1], axis=1)
    b1c = jnp.concatenate([dz[:, 1:], zpad1, zpad1], axis=1)
    g0 = jnp.sum(pdf[:, :, None] * m3, axis=1)
    g1 = cdf[:, 0:1] + jnp.sum(g1c[:, :, None] * m3, axis=1)
    bb0 = z[:, 0:1] + jnp.sum(b0c[:, :, None] * m3, axis=1)
    bb1 = z[:, 1:2] + jnp.sum(b1c[:, :, None] * m3, axis=1)
    denom = g1 - g0
    denom = jnp.where(denom < 1e-5, 1.0, denom)
    tt = (u - g0) / denom
    s = bb0 + tt * (bb1 - bb0)

    big = jnp.full((tr, NS), 3e38, F32)
    c = jnp.concatenate([z, big, s], axis=1)
    lane = lax.broadcasted_iota(I32, (1, 256), 1)
    for k in (128, 64, 32, 16, 8, 4, 2, 1):
        keep = (lane & k) == 0
        left = jnp.concatenate([c[:, k:], c[:, :k]], axis=1)
        right = jnp.concatenate([c[:, 256 - k:], c[:, :256 - k]], axis=1)
        partner = jnp.where(keep, left, right)
        mn = jnp.minimum(c, partner)
        mx = jnp.maximum(c, partner)
        c = jnp.where(keep, mn, mx)
    zf_ref[...] = c[:, :NF]


def _r_body(d_ref, zf_ref, r0_ref, r1_ref, r2_ref, sg_ref,
            rgb_ref, depth_ref, acc_ref, wout_ref):
    tr = TR_R
    d = d_ref[...]
    zf = zf_ref[...]
    dnorm = jnp.sqrt(jnp.sum(d * d, axis=1, keepdims=True))
    rgb_map, weights, _ = _render(r0_ref[...], r1_ref[...], r2_ref[...],
                                  sg_ref[...], zf, dnorm, tr, NF)
    rgb_ref[...] = rgb_map
    depth_ref[...] = jnp.sum(weights * zf, axis=1, keepdims=True)
    acc_ref[...] = jnp.sum(weights, axis=1, keepdims=True)
    wout_ref[...] = weights


@jax.jit
def kernel(rays_o, rays_d, bounds, W1, b1, W2, b2):
    t_vals = jnp.linspace(0.0, 1.0, NS, dtype=F32).reshape(1, NS)
    u_vals = jnp.linspace(0.0, 1.0, NI, dtype=F32)[::-1].reshape(1, NI)
    w1t = W1.T
    w2t = W2.T
    b1c = b1.reshape(HID, 1)
    b2c = b2.reshape(4, 1)

    m1 = NRAYS * NS
    blk1 = TRM1 * NS
    m2 = NRAYS * NF
    blk2 = TRM2 * NF

    def col_spec(rows, cols):
        return pl.BlockSpec((rows, cols), lambda j: (0, j))

    def fix_spec(shape):
        return pl.BlockSpec(shape, lambda j: (0, 0))

    def ray_spec(tr, cols):
        return pl.BlockSpec((tr, cols), lambda j: (j, 0))

    orep1 = jnp.repeat(rays_o.T, NS, axis=1)
    drep1 = jnp.repeat(rays_d.T, NS, axis=1)
    bndrep1 = jnp.repeat(bounds.T, NS, axis=1)
    ttile1 = jnp.tile(t_vals, (1, TRM1))
    orep2 = jnp.repeat(rays_o.T, NF, axis=1)
    drep2 = jnp.repeat(rays_d.T, NF, axis=1)

    rawt1 = pl.pallas_call(
        _m1_body,
        grid=(NRAYS // TRM1,),
        in_specs=[
            col_spec(3, blk1), col_spec(3, blk1), col_spec(2, blk1),
            fix_spec((1, blk1)), fix_spec((HID, 3)), fix_spec((HID, 1)),
            fix_spec((4, HID)), fix_spec((4, 1)),
        ],
        out_specs=col_spec(4, blk1),
        out_shape=jax.ShapeDtypeStruct((4, m1), F32),
    )(orep1, drep1, bndrep1, ttile1, w1t, b1c, w2t, b2c)
    ch1 = [rawt1[c].reshape(NRAYS, NS) for c in range(4)]

    rgbc, zfine = pl.pallas_call(
        _s_body,
        grid=(NRAYS // TR_S,),
        in_specs=[
            ray_spec(TR_S, 3), ray_spec(TR_S, 2),
            fix_spec((1, NS)), fix_spec((1, NI)),
            ray_spec(TR_S, NS), ray_spec(TR_S, NS), ray_spec(TR_S, NS),
            ray_spec(TR_S, NS),
        ],
        out_specs=[ray_spec(TR_S, 3), ray_spec(TR_S, NF)],
        out_shape=[
            jax.ShapeDtypeStruct((NRAYS, 3), F32),
            jax.ShapeDtypeStruct((NRAYS, NF), F32),
        ],
    )(rays_d, bounds, t_vals, u_vals, *ch1)

    zct = zfine.reshape(1, m2)
    rawt2 = pl.pallas_call(
        _m2_body,
        grid=(NRAYS // TRM2,),
        in_specs=[
            col_spec(3, blk2), col_spec(3, blk2), col_spec(1, blk2),
            fix_spec((HID, 3)), fix_spec((HID, 1)),
            fix_spec((4, HID)), fix_spec((4, 1)),
        ],
        out_specs=col_spec(4, blk2),
        out_shape=jax.ShapeDtypeStruct((4, m2), F32),
    )(orep2, drep2, zct, w1t, b1c, w2t, b2c)
    ch2 = [rawt2[c].reshape(NRAYS, NF) for c in range(4)]

    rgb, depth, acc, weights = pl.pallas_call(
        _r_body,
        grid=(NRAYS // TR_R,),
        in_specs=[
            ray_spec(TR_R, 3), ray_spec(TR_R, NF),
            ray_spec(TR_R, NF), ray_spec(TR_R, NF), ray_spec(TR_R, NF),
            ray_spec(TR_R, NF),
        ],
        out_specs=[ray_spec(TR_R, 3), ray_spec(TR_R, 1), ray_spec(TR_R, 1),
                   ray_spec(TR_R, NF)],
        out_shape=[
            jax.ShapeDtypeStruct((NRAYS, 3), F32),
            jax.ShapeDtypeStruct((NRAYS, 1), F32),
            jax.ShapeDtypeStruct((NRAYS, 1), F32),
            jax.ShapeDtypeStruct((NRAYS, NF), F32),
        ],
    )(rays_d, zfine, *ch2)

    return (rgbc, rgb, depth.reshape(NRAYS), acc.reshape(NRAYS), weights)


# speedup vs baseline: 24.9933x; 1.3832x over previous
import jax
import jax.numpy as jnp
from jax import lax
from jax.experimental import pallas as pl

NRAYS = 4096
NS = 64
NI = 128
NF = NS + NI
HID = 256

TRM1 = 64
TRM2 = 32
TR_S = 128
TR_R = 64

F32 = jnp.float32
I32 = jnp.int32


def _cumsum_lanes(x, n):
    k = 1
    while k < n:
        shifted = jnp.concatenate(
            [jnp.zeros(x.shape[:-1] + (k,), x.dtype), x[..., : n - k]], axis=-1)
        x = x + shifted
        k *= 2
    return x


def _sigmoid(x):
    return 1.0 / (1.0 + jnp.exp(-x))


def _mlp_t(pts_t, w1t, b1c, w2t, b2c):
    h = jnp.maximum(
        jnp.dot(w1t, pts_t, preferred_element_type=F32) + b1c, 0.0)
    return jnp.dot(w2t, h, preferred_element_type=F32) + b2c


def _m1_body(orep_ref, drep_ref, bndrep_ref, ttile_ref, w1t_ref, b1c_ref,
             w2t_ref, b2c_ref, rawt_ref):
    near_rep = bndrep_ref[...][0:1, :]
    far_rep = bndrep_ref[...][1:2, :]
    t_rep = ttile_ref[...]
    zc = near_rep * (1.0 - t_rep) + far_rep * t_rep
    pts_t = orep_ref[...] + drep_ref[...] * zc
    rawt_ref[...] = _mlp_t(pts_t, w1t_ref[...], b1c_ref[...],
                           w2t_ref[...], b2c_ref[...])


def _m2_body(orep_ref, drep_ref, zct_ref, w1t_ref, b1c_ref, w2t_ref, b2c_ref,
             rawt_ref):
    pts_t = orep_ref[...] + drep_ref[...] * zct_ref[...]
    rawt_ref[...] = _mlp_t(pts_t, w1t_ref[...], b1c_ref[...],
                           w2t_ref[...], b2c_ref[...])


def _render(r0, r1, r2, sg, z, dnorm, tr, ns):
    sigma = jnp.maximum(sg, 0.0)
    dz = z[:, 1:] - z[:, :-1]
    dists = jnp.concatenate([dz, jnp.full((tr, 1), 1e10, F32)], axis=1)
    dists = dists * dnorm
    e = jnp.exp(-sigma * dists)
    alpha = 1.0 - e
    lt = jnp.log(e + 1e-10)
    ct_inc = _cumsum_lanes(lt, ns)
    ct_exc = jnp.concatenate([jnp.zeros((tr, 1), F32), ct_inc[:, :-1]], axis=1)
    trans = jnp.exp(ct_exc)
    weights = alpha * trans
    rgb_cols = [jnp.sum(weights * _sigmoid(rc), axis=1, keepdims=True)
                for rc in (r0, r1, r2)]
    rgb_map = jnp.concatenate(rgb_cols, axis=1)
    return rgb_map, weights, dz


def _cumsum_sub(x, n):
    k = 1
    while k < n:
        shifted = jnp.concatenate(
            [jnp.zeros((k,) + x.shape[1:], x.dtype), x[: n - k]], axis=0)
        x = x + shifted
        k *= 2
    return x


def _s_body(d_ref, bnd_ref, t_ref, u_ref, r0_ref, r1_ref, r2_ref, sg_ref,
            rgbc_ref, zf_ref):
    tr = TR_S
    d = d_ref[...]
    near = bnd_ref[...][0:1, :]
    far = bnd_ref[...][1:2, :]
    t = t_ref[...]
    z = near * (1.0 - t) + far * t

    dnorm = jnp.sqrt(jnp.sum(d * d, axis=0, keepdims=True))
    sigma = jnp.maximum(sg_ref[...], 0.0)
    dz = z[1:, :] - z[:-1, :]
    dists = jnp.concatenate([dz, jnp.full((1, tr), 1e10, F32)], axis=0)
    dists = dists * dnorm
    e = jnp.exp(-sigma * dists)
    alpha = 1.0 - e
    lt = jnp.log(e + 1e-10)
    ct_inc = _cumsum_sub(lt, NS)
    ct_exc = jnp.concatenate([jnp.zeros((1, tr), F32), ct_inc[:-1, :]], axis=0)
    trans = jnp.exp(ct_exc)
    weights = alpha * trans
    rgb_rows = [jnp.sum(weights * _sigmoid(rc_ref[...]), axis=0, keepdims=True)
                for rc_ref in (r0_ref, r1_ref, r2_ref)]
    rgbc_ref[...] = jnp.concatenate(rgb_rows, axis=0)

    w = weights + 1e-5
    pdf = w / jnp.sum(w, axis=0, keepdims=True)
    cdf = _cumsum_sub(pdf, NS)
    u = u_ref[...]

    m3 = (cdf[:, None, :] <= u[None, :, :]).astype(F32)
    zpad1 = jnp.zeros((1, tr), F32)
    g1c = jnp.concatenate([pdf[1:, :], zpad1], axis=0)
    b0c = jnp.concatenate([dz, zpad1], axis=0)
    b1c = jnp.concatenate([dz[1:, :], zpad1, zpad1], axis=0)
    g0 = jnp.sum(pdf[:, None, :] * m3, axis=0)
    g1 = cdf[0:1, :] + jnp.sum(g1c[:, None, :] * m3, axis=0)
    bb0 = z[0:1, :] + jnp.sum(b0c[:, None, :] * m3, axis=0)
    bb1 = z[1:2, :] + jnp.sum(b1c[:, None, :] * m3, axis=0)
    denom = g1 - g0
    denom = jnp.where(denom < 1e-5, 1.0, denom)
    tt = (u - g0) / denom
    s = bb0 + tt * (bb1 - bb0)

    big = jnp.full((NS, tr), 3e38, F32)
    c = jnp.concatenate([z, big, s], axis=0)
    row = lax.broadcasted_iota(I32, (256, 1), 0)
    for k in (128, 64, 32, 16, 8, 4, 2, 1):
        keep = (row & k) == 0
        down = jnp.concatenate([c[k:, :], c[:k, :]], axis=0)
        up = jnp.concatenate([c[256 - k:, :], c[:256 - k, :]], axis=0)
        partner = jnp.where(keep, down, up)
        mn = jnp.minimum(c, partner)
        mx = jnp.maximum(c, partner)
        c = jnp.where(keep, mn, mx)
    zf_ref[...] = c[:NF, :]


def _r_body(d_ref, zf_ref, r0_ref, r1_ref, r2_ref, sg_ref,
            rgb_ref, depth_ref, acc_ref, wout_ref):
    tr = TR_R
    d = d_ref[...]
    zf = zf_ref[...]
    dnorm = jnp.sqrt(jnp.sum(d * d, axis=1, keepdims=True))
    rgb_map, weights, _ = _render(r0_ref[...], r1_ref[...], r2_ref[...],
                                  sg_ref[...], zf, dnorm, tr, NF)
    rgb_ref[...] = rgb_map
    depth_ref[...] = jnp.sum(weights * zf, axis=1, keepdims=True)
    acc_ref[...] = jnp.sum(weights, axis=1, keepdims=True)
    wout_ref[...] = weights


@jax.jit
def kernel(rays_o, rays_d, bounds, W1, b1, W2, b2):
    t_vals = jnp.linspace(0.0, 1.0, NS, dtype=F32).reshape(1, NS)
    t_col = t_vals.reshape(NS, 1)
    u_col = jnp.linspace(0.0, 1.0, NI, dtype=F32)[::-1].reshape(NI, 1)
    w1t = W1.T
    w2t = W2.T
    b1c = b1.reshape(HID, 1)
    b2c = b2.reshape(4, 1)

    m1 = NRAYS * NS
    blk1 = TRM1 * NS
    m2 = NRAYS * NF
    blk2 = TRM2 * NF

    def col_spec(rows, cols):
        return pl.BlockSpec((rows, cols), lambda j: (0, j))

    def fix_spec(shape):
        return pl.BlockSpec(shape, lambda j: (0, 0))

    def ray_spec(tr, cols):
        return pl.BlockSpec((tr, cols), lambda j: (j, 0))

    orep1 = jnp.repeat(rays_o.T, NS, axis=1)
    drep1 = jnp.repeat(rays_d.T, NS, axis=1)
    bndrep1 = jnp.repeat(bounds.T, NS, axis=1)
    ttile1 = jnp.tile(t_vals, (1, TRM1))
    orep2 = jnp.repeat(rays_o.T, NF, axis=1)
    drep2 = jnp.repeat(rays_d.T, NF, axis=1)

    rawt1 = pl.pallas_call(
        _m1_body,
        grid=(NRAYS // TRM1,),
        in_specs=[
            col_spec(3, blk1), col_spec(3, blk1), col_spec(2, blk1),
            fix_spec((1, blk1)), fix_spec((HID, 3)), fix_spec((HID, 1)),
            fix_spec((4, HID)), fix_spec((4, 1)),
        ],
        out_specs=col_spec(4, blk1),
        out_shape=jax.ShapeDtypeStruct((4, m1), F32),
    )(orep1, drep1, bndrep1, ttile1, w1t, b1c, w2t, b2c)
    ch1t = [rawt1[c].reshape(NRAYS, NS).T for c in range(4)]

    rgbct, zfinet = pl.pallas_call(
        _s_body,
        grid=(NRAYS // TR_S,),
        in_specs=[
            col_spec(3, TR_S), col_spec(2, TR_S),
            fix_spec((NS, 1)), fix_spec((NI, 1)),
            col_spec(NS, TR_S), col_spec(NS, TR_S), col_spec(NS, TR_S),
            col_spec(NS, TR_S),
        ],
        out_specs=[col_spec(3, TR_S), col_spec(NF, TR_S)],
        out_shape=[
            jax.ShapeDtypeStruct((3, NRAYS), F32),
            jax.ShapeDtypeStruct((NF, NRAYS), F32),
        ],
    )(rays_d.T, bounds.T, t_col, u_col, *ch1t)
    rgbc = rgbct.T
    zfine = zfinet.T

    zct = zfine.reshape(1, m2)
    rawt2 = pl.pallas_call(
        _m2_body,
        grid=(NRAYS // TRM2,),
        in_specs=[
            col_spec(3, blk2), col_spec(3, blk2), col_spec(1, blk2),
            fix_spec((HID, 3)), fix_spec((HID, 1)),
            fix_spec((4, HID)), fix_spec((4, 1)),
        ],
        out_specs=col_spec(4, blk2),
        out_shape=jax.ShapeDtypeStruct((4, m2), F32),
    )(orep2, drep2, zct, w1t, b1c, w2t, b2c)
    ch2 = [rawt2[c].reshape(NRAYS, NF) for c in range(4)]

    rgb, depth, acc, weights = pl.pallas_call(
        _r_body,
        grid=(NRAYS // TR_R,),
        in_specs=[
            ray_spec(TR_R, 3), ray_spec(TR_R, NF),
            ray_spec(TR_R, NF), ray_spec(TR_R, NF), ray_spec(TR_R, NF),
            ray_spec(TR_R, NF),
        ],
        out_specs=[ray_spec(TR_R, 3), ray_spec(TR_R, 1), ray_spec(TR_R, 1),
                   ray_spec(TR_R, NF)],
        out_shape=[
            jax.ShapeDtypeStruct((NRAYS, 3), F32),
            jax.ShapeDtypeStruct((NRAYS, 1), F32),
            jax.ShapeDtypeStruct((NRAYS, 1), F32),
            jax.ShapeDtypeStruct((NRAYS, NF), F32),
        ],
    )(rays_d, zfine, *ch2)

    return (rgbc, rgb, depth.reshape(NRAYS), acc.reshape(NRAYS), weights)


# speedup vs baseline: 35.0800x; 1.4036x over previous
import jax
import jax.numpy as jnp
from jax import lax
from jax.experimental import pallas as pl

NRAYS = 4096
NS = 64
NI = 128
NF = NS + NI
HID = 256

SPG1 = 4
SPG2 = 4
TR_S = 128
TR_R = 128

F32 = jnp.float32
I32 = jnp.int32


def _cumsum_lanes(x, n):
    k = 1
    while k < n:
        shifted = jnp.concatenate(
            [jnp.zeros(x.shape[:-1] + (k,), x.dtype), x[..., : n - k]], axis=-1)
        x = x + shifted
        k *= 2
    return x


def _sigmoid(x):
    return 1.0 / (1.0 + jnp.exp(-x))


def _mlp_t(pts_t, w1t, b1c, w2t, b2c):
    h = jnp.maximum(
        jnp.dot(w1t, pts_t, preferred_element_type=F32) + b1c, 0.0)
    return jnp.dot(w2t, h, preferred_element_type=F32) + b2c


def _m1_body(ot_ref, dt_ref, bndt_ref, t_ref, w1t_ref, b1c_ref,
             w2t_ref, b2c_ref, rawt_ref):
    ot = ot_ref[...]
    dt = dt_ref[...]
    near = bndt_ref[...][0:1, :]
    far = bndt_ref[...][1:2, :]
    tv = jnp.squeeze(t_ref[...], axis=1)
    cols = []
    for g in range(SPG1):
        tg = tv[g:g + 1, :]
        zc = near * (1.0 - tg) + far * tg
        cols.append(ot + dt * zc)
    pts_t = jnp.concatenate(cols, axis=1)
    rawt_ref[...] = _mlp_t(pts_t, w1t_ref[...], b1c_ref[...],
                           w2t_ref[...], b2c_ref[...])


def _m2_body(ot_ref, dt_ref, zct_ref, w1t_ref, b1c_ref, w2t_ref, b2c_ref,
             rawt_ref):
    ot = ot_ref[...]
    dt = dt_ref[...]
    zc = jnp.squeeze(zct_ref[...], axis=1)
    cols = [ot + dt * zc[g:g + 1, :] for g in range(SPG2)]
    pts_t = jnp.concatenate(cols, axis=1)
    rawt_ref[...] = _mlp_t(pts_t, w1t_ref[...], b1c_ref[...],
                           w2t_ref[...], b2c_ref[...])


def _render(r0, r1, r2, sg, z, dnorm, tr, ns):
    sigma = jnp.maximum(sg, 0.0)
    dz = z[:, 1:] - z[:, :-1]
    dists = jnp.concatenate([dz, jnp.full((tr, 1), 1e10, F32)], axis=1)
    dists = dists * dnorm
    e = jnp.exp(-sigma * dists)
    alpha = 1.0 - e
    lt = jnp.log(e + 1e-10)
    ct_inc = _cumsum_lanes(lt, ns)
    ct_exc = jnp.concatenate([jnp.zeros((tr, 1), F32), ct_inc[:, :-1]], axis=1)
    trans = jnp.exp(ct_exc)
    weights = alpha * trans
    rgb_cols = [jnp.sum(weights * _sigmoid(rc), axis=1, keepdims=True)
                for rc in (r0, r1, r2)]
    rgb_map = jnp.concatenate(rgb_cols, axis=1)
    return rgb_map, weights, dz


def _cumsum_sub(x, n):
    k = 1
    while k < n:
        shifted = jnp.concatenate(
            [jnp.zeros((k,) + x.shape[1:], x.dtype), x[: n - k]], axis=0)
        x = x + shifted
        k *= 2
    return x


def _s_body(d_ref, bnd_ref, t_ref, u_ref, r0_ref, r1_ref, r2_ref, sg_ref,
            rgbc_ref, zf_ref):
    tr = TR_S
    d = d_ref[...]
    near = bnd_ref[...][0:1, :]
    far = bnd_ref[...][1:2, :]
    t = t_ref[...]
    z = near * (1.0 - t) + far * t

    dnorm = jnp.sqrt(jnp.sum(d * d, axis=0, keepdims=True))
    sigma = jnp.maximum(sg_ref[...], 0.0)
    dz = z[1:, :] - z[:-1, :]
    dists = jnp.concatenate([dz, jnp.full((1, tr), 1e10, F32)], axis=0)
    dists = dists * dnorm
    e = jnp.exp(-sigma * dists)
    alpha = 1.0 - e
    lt = jnp.log(e + 1e-10)
    ct_inc = _cumsum_sub(lt, NS)
    ct_exc = jnp.concatenate([jnp.zeros((1, tr), F32), ct_inc[:-1, :]], axis=0)
    trans = jnp.exp(ct_exc)
    weights = alpha * trans
    rgb_rows = [jnp.sum(weights * _sigmoid(rc_ref[...]), axis=0, keepdims=True)
                for rc_ref in (r0_ref, r1_ref, r2_ref)]
    rgbc_ref[...] = jnp.concatenate(rgb_rows, axis=0)

    w = weights + 1e-5
    pdf = w / jnp.sum(w, axis=0, keepdims=True)
    cdf = _cumsum_sub(pdf, NS)
    u = u_ref[...]

    m3 = (cdf[:, None, :] <= u[None, :, :]).astype(F32)
    zpad1 = jnp.zeros((1, tr), F32)
    g1c = jnp.concatenate([pdf[1:, :], zpad1], axis=0)
    b0c = jnp.concatenate([dz, zpad1], axis=0)
    b1c = jnp.concatenate([dz[1:, :], zpad1, zpad1], axis=0)
    g0 = jnp.sum(pdf[:, None, :] * m3, axis=0)
    g1 = cdf[0:1, :] + jnp.sum(g1c[:, None, :] * m3, axis=0)
    bb0 = z[0:1, :] + jnp.sum(b0c[:, None, :] * m3, axis=0)
    bb1 = z[1:2, :] + jnp.sum(b1c[:, None, :] * m3, axis=0)
    denom = g1 - g0
    denom = jnp.where(denom < 1e-5, 1.0, denom)
    tt = (u - g0) / denom
    s = bb0 + tt * (bb1 - bb0)

    big = jnp.full((NS, tr), 3e38, F32)
    c = jnp.concatenate([z, big, s], axis=0)
    row = lax.broadcasted_iota(I32, (256, 1), 0)
    for k in (128, 64, 32, 16, 8, 4, 2, 1):
        keep = (row & k) == 0
        down = jnp.concatenate([c[k:, :], c[:k, :]], axis=0)
        up = jnp.concatenate([c[256 - k:, :], c[:256 - k, :]], axis=0)
        partner = jnp.where(keep, down, up)
        mn = jnp.minimum(c, partner)
        mx = jnp.maximum(c, partner)
        c = jnp.where(keep, mn, mx)
    zf_ref[...] = c[:NF, :]


def _r_body(d_ref, zf_ref, r0_ref, r1_ref, r2_ref, sg_ref,
            rgb_ref, depth_ref, acc_ref, wout_ref):
    tr = TR_R
    d = d_ref[...]
    zf = zf_ref[...]
    dnorm = jnp.sqrt(jnp.sum(d * d, axis=0, keepdims=True))
    sigma = jnp.maximum(sg_ref[...], 0.0)
    dz = zf[1:, :] - zf[:-1, :]
    dists = jnp.concatenate([dz, jnp.full((1, tr), 1e10, F32)], axis=0)
    dists = dists * dnorm
    e = jnp.exp(-sigma * dists)
    alpha = 1.0 - e
    lt = jnp.log(e + 1e-10)
    ct_inc = _cumsum_sub(lt, NF)
    ct_exc = jnp.concatenate([jnp.zeros((1, tr), F32), ct_inc[:-1, :]], axis=0)
    trans = jnp.exp(ct_exc)
    weights = alpha * trans
    rgb_rows = [jnp.sum(weights * _sigmoid(rc_ref[...]), axis=0, keepdims=True)
                for rc_ref in (r0_ref, r1_ref, r2_ref)]
    rgb_ref[...] = jnp.concatenate(rgb_rows, axis=0)
    depth_ref[...] = jnp.sum(weights * zf, axis=0, keepdims=True)
    acc_ref[...] = jnp.sum(weights, axis=0, keepdims=True)
    wout_ref[...] = weights


@jax.jit
def kernel(rays_o, rays_d, bounds, W1, b1, W2, b2):
    t_vals = jnp.linspace(0.0, 1.0, NS, dtype=F32).reshape(1, NS)
    t_col = t_vals.reshape(NS, 1)
    u_col = jnp.linspace(0.0, 1.0, NI, dtype=F32)[::-1].reshape(NI, 1)
    w1t = W1.T
    w2t = W2.T
    b1c = b1.reshape(HID, 1)
    b2c = b2.reshape(4, 1)

    m1 = NRAYS * NS
    m2 = NRAYS * NF
    ot = rays_o.T
    dt = rays_d.T
    bndt = bounds.T

    def col_spec(rows, cols):
        return pl.BlockSpec((rows, cols), lambda j: (0, j))

    def fix_spec(shape):
        return pl.BlockSpec(shape, lambda j: (0, 0))

    rawt1 = pl.pallas_call(
        _m1_body,
        grid=(NS // SPG1,),
        in_specs=[
            fix_spec((3, NRAYS)), fix_spec((3, NRAYS)), fix_spec((2, NRAYS)),
            pl.BlockSpec((SPG1, 1, 1), lambda j: (j, 0, 0)),
            fix_spec((HID, 3)), fix_spec((HID, 1)),
            fix_spec((4, HID)), fix_spec((4, 1)),
        ],
        out_specs=col_spec(4, SPG1 * NRAYS),
        out_shape=jax.ShapeDtypeStruct((4, m1), F32),
    )(ot, dt, bndt, t_col.reshape(NS, 1, 1), w1t, b1c, w2t, b2c)
    ch1t = [rawt1[c].reshape(NS, NRAYS) for c in range(4)]

    rgbct, zfinet = pl.pallas_call(
        _s_body,
        grid=(NRAYS // TR_S,),
        in_specs=[
            col_spec(3, TR_S), col_spec(2, TR_S),
            fix_spec((NS, 1)), fix_spec((NI, 1)),
            col_spec(NS, TR_S), col_spec(NS, TR_S), col_spec(NS, TR_S),
            col_spec(NS, TR_S),
        ],
        out_specs=[col_spec(3, TR_S), col_spec(NF, TR_S)],
        out_shape=[
            jax.ShapeDtypeStruct((3, NRAYS), F32),
            jax.ShapeDtypeStruct((NF, NRAYS), F32),
        ],
    )(dt, bndt, t_col, u_col, *ch1t)

    rawt2 = pl.pallas_call(
        _m2_body,
        grid=(NF // SPG2,),
        in_specs=[
            fix_spec((3, NRAYS)), fix_spec((3, NRAYS)),
            pl.BlockSpec((SPG2, 1, NRAYS), lambda j: (j, 0, 0)),
            fix_spec((HID, 3)), fix_spec((HID, 1)),
            fix_spec((4, HID)), fix_spec((4, 1)),
        ],
        out_specs=col_spec(4, SPG2 * NRAYS),
        out_shape=jax.ShapeDtypeStruct((4, m2), F32),
    )(ot, dt, zfinet.reshape(NF, 1, NRAYS), w1t, b1c, w2t, b2c)
    ch2t = [rawt2[c].reshape(NF, NRAYS) for c in range(4)]

    rgbt, deptht, acct, weightst = pl.pallas_call(
        _r_body,
        grid=(NRAYS // TR_R,),
        in_specs=[
            col_spec(3, TR_R), col_spec(NF, TR_R),
            col_spec(NF, TR_R), col_spec(NF, TR_R), col_spec(NF, TR_R),
            col_spec(NF, TR_R),
        ],
        out_specs=[col_spec(3, TR_R), col_spec(1, TR_R), col_spec(1, TR_R),
                   col_spec(NF, TR_R)],
        out_shape=[
            jax.ShapeDtypeStruct((3, NRAYS), F32),
            jax.ShapeDtypeStruct((1, NRAYS), F32),
            jax.ShapeDtypeStruct((1, NRAYS), F32),
            jax.ShapeDtypeStruct((NF, NRAYS), F32),
        ],
    )(dt, zfinet, *ch2t)

    return (rgbct.T, rgbt.T, deptht.reshape(NRAYS), acct.reshape(NRAYS),
            weightst.T)
